# baseline (speedup 1.0000x reference)
import functools

import jax
import jax.numpy as jnp
from jax import lax
from jax.experimental import pallas as pl
from jax.experimental.pallas import tpu as pltpu
from jax.experimental.pallas import tpu_sc as plsc

NUM_CORES = 2
NUM_SUBCORES = 16
NW = NUM_CORES * NUM_SUBCORES
CH = 128
LANES = 16
COMBO_REP = 1024


def _sc_gather(tok_table, combo_table, ids_2d, seg_2d, n_chunks, hidden):
    per_w = n_chunks * CH
    mesh = plsc.VectorSubcoreMesh(
        core_axis_name="c", subcore_axis_name="s",
        num_cores=NUM_CORES, num_subcores=NUM_SUBCORES)

    @functools.partial(
        pl.kernel,
        out_type=jax.ShapeDtypeStruct((NW, n_chunks, CH, hidden), jnp.float32),
        mesh=mesh,
        scratch_types=[
            pltpu.VMEM((per_w,), jnp.int32),
            pltpu.VMEM((per_w,), jnp.int32),
            pltpu.VMEM((CH, hidden), jnp.float32),
            pltpu.SemaphoreType.DMA,
            pltpu.SemaphoreType.DMA,
        ],
    )
    def gather_kernel(table_hbm, combo_hbm, ids_hbm, seg_hbm, out_hbm,
                      idx_v, cmb_v, buf0, gsem, asem):
        wid = lax.axis_index("s") * NUM_CORES + lax.axis_index("c")
        pltpu.sync_copy(ids_hbm.at[wid], idx_v)
        pltpu.sync_copy(seg_hbm.at[wid], cmb_v)

        def mkcombo(t, _):
            ids16 = idx_v[pl.ds(t * LANES, LANES)]
            seg16 = cmb_v[pl.ds(t * LANES, LANES)]
            flat16 = wid * per_w + t * LANES + lax.iota(jnp.int32, LANES)
            spread = lax.bitwise_and(flat16, COMBO_REP - 1) * 8
            cmb_v[pl.ds(t * LANES, LANES)] = spread + seg16 + jnp.where(
                ids16 == 0, 2, 0)
            return 0

        lax.fori_loop(0, per_w // LANES, mkcombo, 0)

        def body(c, _):
            rows = pl.ds(c * CH, CH)
            pltpu.async_copy(
                table_hbm.at[idx_v.at[rows]], buf0, gsem).wait()
            pltpu.async_copy(
                combo_hbm.at[cmb_v.at[rows]], buf0, asem, add=True).wait()
            pltpu.sync_copy(buf0, out_hbm.at[wid, c])
            return 0

        lax.fori_loop(0, n_chunks, body, 0)

    return gather_kernel(tok_table, combo_table, ids_2d, seg_2d)


def _tc_finish(embed, pos_slab, gb, rows_blk):
    n, hidden = embed.shape
    grid = (n // rows_blk,)

    def body(emb_ref, pos_ref, gb_ref, out_ref):
        x = emb_ref[...] + pos_ref[...]
        mean = jnp.mean(x, axis=-1, keepdims=True)
        xc = x - mean
        var = jnp.mean(xc * xc, axis=-1, keepdims=True)
        y = xc * lax.rsqrt(var + 1e-5)
        gb = gb_ref[...]
        out_ref[...] = y * gb[0:1, :] + gb[1:2, :]

    return pl.pallas_call(
        body,
        grid=grid,
        in_specs=[
            pl.BlockSpec((rows_blk, hidden), lambda i: (i, 0)),
            pl.BlockSpec((rows_blk, hidden), lambda i: (0, 0)),
            pl.BlockSpec((8, hidden), lambda i: (0, 0)),
        ],
        out_specs=pl.BlockSpec((rows_blk, hidden), lambda i: (i, 0)),
        out_shape=jax.ShapeDtypeStruct((n, hidden), jnp.float32),
    )(embed, pos_slab, gb)


def kernel(input_ids, segment_ids, tok_table, pos_table, seg_table, gamma,
           beta):
    bsz, sent_len = input_ids.shape
    hidden = tok_table.shape[1]
    n_tok = bsz * sent_len
    assert n_tok % (NW * CH) == 0
    n_chunks = n_tok // (NW * CH)
    per_w = n_chunks * CH

    tok0 = tok_table[0:1]
    combo_table = jnp.concatenate(
        [seg_table[0:1], seg_table[1:2],
         seg_table[0:1] - tok0, seg_table[1:2] - tok0,
         jnp.zeros((4, hidden), jnp.float32)], axis=0)
    combo_table = jnp.tile(combo_table, (COMBO_REP, 1))

    ids_2d = input_ids.reshape(NW, per_w)
    seg_2d = segment_ids.reshape(NW, per_w)
    gathered = _sc_gather(tok_table, combo_table, ids_2d, seg_2d,
                          n_chunks, hidden)

    rows_blk = 4 * sent_len
    pos_slab = jnp.tile(pos_table[:sent_len], (4, 1))
    gb = jnp.concatenate(
        [gamma[None, :], beta[None, :], jnp.zeros((6, hidden), jnp.float32)],
        axis=0)
    out = _tc_finish(gathered.reshape(n_tok, hidden), pos_slab, gb, rows_blk)
    return out.reshape(bsz, sent_len, hidden)


# speedup vs baseline: 1.0129x; 1.0129x over previous
import functools

import jax
import jax.numpy as jnp
from jax import lax
from jax.experimental import pallas as pl
from jax.experimental.pallas import tpu as pltpu
from jax.experimental.pallas import tpu_sc as plsc

NUM_CORES = 2
NUM_SUBCORES = 16
NW = NUM_CORES * NUM_SUBCORES
CH = 128


def _sc_gather(tok_table, ids_2d, n_chunks, hidden):
    mesh = plsc.VectorSubcoreMesh(
        core_axis_name="c", subcore_axis_name="s",
        num_cores=NUM_CORES, num_subcores=NUM_SUBCORES)

    @functools.partial(
        pl.kernel,
        out_type=jax.ShapeDtypeStruct((NW, n_chunks, CH, hidden), jnp.float32),
        mesh=mesh,
        scratch_types=[
            pltpu.VMEM((n_chunks * CH,), jnp.int32),
            pltpu.VMEM((CH, hidden), jnp.float32),
            pltpu.SemaphoreType.DMA,
        ],
    )
    def gather_kernel(table_hbm, ids_hbm, out_hbm, idx_v, buf0, gsem):
        wid = lax.axis_index("s") * NUM_CORES + lax.axis_index("c")
        pltpu.sync_copy(ids_hbm.at[wid], idx_v)

        def body(c, _):
            rows = pl.ds(c * CH, CH)
            pltpu.async_copy(
                table_hbm.at[idx_v.at[rows]], buf0, gsem).wait()
            pltpu.sync_copy(buf0, out_hbm.at[wid, c])
            return 0

        lax.fori_loop(0, n_chunks, body, 0)

    return gather_kernel(tok_table, ids_2d)


def _tc_finish(embed, onehot, pos_slab, combo8, gb, rows_blk):
    n, hidden = embed.shape

    def body(emb_ref, oh_ref, pos_ref, combo_ref, gb_ref, out_ref):
        x = emb_ref[...] + pos_ref[...]
        x = x + jnp.dot(oh_ref[...], combo_ref[...],
                        preferred_element_type=jnp.float32,
                        precision=lax.Precision.HIGHEST)
        mean = jnp.mean(x, axis=-1, keepdims=True)
        xc = x - mean
        var = jnp.mean(xc * xc, axis=-1, keepdims=True)
        y = xc * lax.rsqrt(var + 1e-5)
        gbv = gb_ref[...]
        out_ref[...] = y * gbv[0:1, :] + gbv[1:2, :]

    return pl.pallas_call(
        body,
        grid=(n // rows_blk,),
        in_specs=[
            pl.BlockSpec((rows_blk, hidden), lambda i: (i, 0)),
            pl.BlockSpec((rows_blk, 8), lambda i: (i, 0)),
            pl.BlockSpec((rows_blk, hidden), lambda i: (0, 0)),
            pl.BlockSpec((8, hidden), lambda i: (0, 0)),
            pl.BlockSpec((8, hidden), lambda i: (0, 0)),
        ],
        out_specs=pl.BlockSpec((rows_blk, hidden), lambda i: (i, 0)),
        out_shape=jax.ShapeDtypeStruct((n, hidden), jnp.float32),
    )(embed, onehot, pos_slab, combo8, gb)


def kernel(input_ids, segment_ids, tok_table, pos_table, seg_table, gamma,
           beta):
    bsz, sent_len = input_ids.shape
    hidden = tok_table.shape[1]
    n_tok = bsz * sent_len
    assert n_tok % (NW * CH) == 0
    n_chunks = n_tok // (NW * CH)
    per_w = n_chunks * CH

    ids_2d = input_ids.reshape(NW, per_w)
    gathered = _sc_gather(tok_table, ids_2d, n_chunks, hidden)

    tok0 = tok_table[0:1]
    combo8 = jnp.concatenate(
        [seg_table[0:1], seg_table[1:2],
         seg_table[0:1] - tok0, seg_table[1:2] - tok0,
         jnp.zeros((4, hidden), jnp.float32)], axis=0)
    combo_idx = (segment_ids + 2 * (input_ids == 0)).reshape(n_tok)
    onehot = (combo_idx[:, None] == jnp.arange(8)[None, :]).astype(
        jnp.float32)

    rows_blk = 4 * sent_len
    pos_slab = jnp.tile(pos_table[:sent_len], (4, 1))
    gb = jnp.concatenate(
        [gamma[None, :], beta[None, :], jnp.zeros((6, hidden), jnp.float32)],
        axis=0)
    out = _tc_finish(gathered.reshape(n_tok, hidden), onehot, pos_slab,
                     combo8, gb, rows_blk)
    return out.reshape(bsz, sent_len, hidden)


# speedup vs baseline: 1.3926x; 1.3748x over previous
import functools

import jax
import jax.numpy as jnp
from jax import lax
from jax.experimental import pallas as pl
from jax.experimental.pallas import tpu as pltpu
from jax.experimental.pallas import tpu_sc as plsc

NUM_CORES = 2
NUM_SUBCORES = 16
NW = NUM_CORES * NUM_SUBCORES
CH = 128


def _sc_gather(tok_table, ids_2d, n_chunks, hidden):
    mesh = plsc.VectorSubcoreMesh(
        core_axis_name="c", subcore_axis_name="s",
        num_cores=NUM_CORES, num_subcores=NUM_SUBCORES)

    @functools.partial(
        pl.kernel,
        out_type=jax.ShapeDtypeStruct((NW, n_chunks, CH, hidden), jnp.float32),
        mesh=mesh,
        scratch_types=[
            pltpu.VMEM((n_chunks * CH,), jnp.int32),
            pltpu.VMEM((CH, hidden), jnp.float32),
            pltpu.SemaphoreType.DMA,
        ],
    )
    def gather_kernel(table_hbm, ids_hbm, out_hbm, idx_v, buf0, gsem):
        wid = lax.axis_index("s") * NUM_CORES + lax.axis_index("c")
        pltpu.sync_copy(ids_hbm.at[wid], idx_v)

        def body(c, _):
            rows = pl.ds(c * CH, CH)
            pltpu.async_copy(
                table_hbm.at[idx_v.at[rows]], buf0, gsem).wait()
            pltpu.sync_copy(buf0, out_hbm.at[wid, c])
            return 0

        lax.fori_loop(0, n_chunks, body, 0)

    return gather_kernel(tok_table, ids_2d)


def _tc_finish(embed, onehot, pos_slab, combo8, gb, rows_blk):
    n, hidden = embed.shape

    def body(emb_ref, oh_ref, pos_ref, combo_ref, gb_ref, out_ref):
        x = emb_ref[...] + pos_ref[...]
        x = x + jnp.dot(oh_ref[...], combo_ref[...],
                        preferred_element_type=jnp.float32,
                        precision=lax.Precision.HIGHEST)
        mean = jnp.mean(x, axis=-1, keepdims=True)
        xc = x - mean
        var = jnp.mean(xc * xc, axis=-1, keepdims=True)
        y = xc * lax.rsqrt(var + 1e-5)
        gbv = gb_ref[...]
        out_ref[...] = y * gbv[0:1, :] + gbv[1:2, :]

    return pl.pallas_call(
        body,
        grid=(n // rows_blk,),
        in_specs=[
            pl.BlockSpec((rows_blk, hidden), lambda i: (i, 0)),
            pl.BlockSpec((rows_blk, 8), lambda i: (i, 0)),
            pl.BlockSpec((rows_blk, hidden), lambda i: (0, 0)),
            pl.BlockSpec((8, hidden), lambda i: (0, 0)),
            pl.BlockSpec((8, hidden), lambda i: (0, 0)),
        ],
        out_specs=pl.BlockSpec((rows_blk, hidden), lambda i: (i, 0)),
        out_shape=jax.ShapeDtypeStruct((n, hidden), jnp.float32),
    )(embed, onehot, pos_slab, combo8, gb)


def kernel(input_ids, segment_ids, tok_table, pos_table, seg_table, gamma,
           beta):
    bsz, sent_len = input_ids.shape
    hidden = tok_table.shape[1]
    n_tok = bsz * sent_len
    assert n_tok % (NW * CH) == 0
    n_chunks = n_tok // (NW * CH)
    per_w = n_chunks * CH

    ids_2d = input_ids.reshape(NW, per_w)
    gathered = _sc_gather(tok_table, ids_2d, n_chunks, hidden)

    tok0 = tok_table[0:1]
    combo8 = jnp.concatenate(
        [seg_table[0:1], seg_table[1:2],
         seg_table[0:1] - tok0, seg_table[1:2] - tok0,
         jnp.zeros((4, hidden), jnp.float32)], axis=0)
    combo_idx = (segment_ids + 2 * (input_ids == 0)).reshape(n_tok)
    onehot = (combo_idx[:, None] == jnp.arange(8)[None, :]).astype(
        jnp.float32)

    rows_blk = 16 * sent_len
    pos_slab = jnp.tile(pos_table[:sent_len], (16, 1))
    gb = jnp.concatenate(
        [gamma[None, :], beta[None, :], jnp.zeros((6, hidden), jnp.float32)],
        axis=0)
    out = _tc_finish(gathered.reshape(n_tok, hidden), onehot, pos_slab,
                     combo8, gb, rows_blk)
    return out.reshape(bsz, sent_len, hidden)


# speedup vs baseline: 1.4654x; 1.0523x over previous
import functools

import jax
import jax.numpy as jnp
from jax import lax
from jax.experimental import pallas as pl
from jax.experimental.pallas import tpu as pltpu
from jax.experimental.pallas import tpu_sc as plsc

NUM_CORES = 2
NUM_SUBCORES = 16
NW = NUM_CORES * NUM_SUBCORES
CH = 128


def _sc_gather(tok_table, ids_2d, n_chunks, hidden):
    mesh = plsc.VectorSubcoreMesh(
        core_axis_name="c", subcore_axis_name="s",
        num_cores=NUM_CORES, num_subcores=NUM_SUBCORES)

    @functools.partial(
        pl.kernel,
        out_type=jax.ShapeDtypeStruct((NW, n_chunks, CH, hidden), jnp.float32),
        mesh=mesh,
        scratch_types=[
            pltpu.VMEM((n_chunks * CH,), jnp.int32),
            pltpu.VMEM((CH, hidden), jnp.float32),
            pltpu.SemaphoreType.DMA,
        ],
    )
    def gather_kernel(table_hbm, ids_hbm, out_hbm, idx_v, buf0, gsem):
        wid = lax.axis_index("s") * NUM_CORES + lax.axis_index("c")
        pltpu.sync_copy(ids_hbm.at[wid], idx_v)

        def body(c, _):
            rows = pl.ds(c * CH, CH)
            pltpu.async_copy(
                table_hbm.at[idx_v.at[rows]], buf0, gsem).wait()
            pltpu.sync_copy(buf0, out_hbm.at[wid, c])
            return 0

        lax.fori_loop(0, n_chunks, body, 0)

    return gather_kernel(tok_table, ids_2d)


def _tc_finish(embed, onehot, pos_slab, combo8, gb, rows_blk):
    n, hidden = embed.shape

    def body(emb_ref, oh_ref, pos_ref, combo_ref, gb_ref, out_ref):
        x = emb_ref[...] + pos_ref[...]
        x = x + jnp.dot(oh_ref[...], combo_ref[...],
                        preferred_element_type=jnp.float32,
                        precision=lax.Precision.HIGHEST)
        mean = jnp.mean(x, axis=-1, keepdims=True)
        xc = x - mean
        var = jnp.mean(xc * xc, axis=-1, keepdims=True)
        y = xc * lax.rsqrt(var + 1e-5)
        gbv = gb_ref[...]
        out_ref[...] = y * gbv[0:1, :] + gbv[1:2, :]

    return pl.pallas_call(
        body,
        grid=(n // rows_blk,),
        in_specs=[
            pl.BlockSpec((rows_blk, hidden), lambda i: (i, 0)),
            pl.BlockSpec((rows_blk, 8), lambda i: (i, 0)),
            pl.BlockSpec((rows_blk, hidden), lambda i: (0, 0)),
            pl.BlockSpec((8, hidden), lambda i: (0, 0)),
            pl.BlockSpec((8, hidden), lambda i: (0, 0)),
        ],
        out_specs=pl.BlockSpec((rows_blk, hidden), lambda i: (i, 0)),
        out_shape=jax.ShapeDtypeStruct((n, hidden), jnp.float32),
    )(embed, onehot, pos_slab, combo8, gb)


def kernel(input_ids, segment_ids, tok_table, pos_table, seg_table, gamma,
           beta):
    bsz, sent_len = input_ids.shape
    hidden = tok_table.shape[1]
    n_tok = bsz * sent_len
    assert n_tok % (NW * CH) == 0
    n_chunks = n_tok // (NW * CH)
    per_w = n_chunks * CH

    ids_2d = input_ids.reshape(NW, per_w)
    gathered = _sc_gather(tok_table, ids_2d, n_chunks, hidden)

    tok0 = tok_table[0:1]
    combo8 = jnp.concatenate(
        [seg_table[0:1], seg_table[1:2],
         seg_table[0:1] - tok0, seg_table[1:2] - tok0,
         jnp.zeros((4, hidden), jnp.float32)], axis=0)
    combo_idx = (segment_ids + 2 * (input_ids == 0)).reshape(n_tok)
    onehot = (combo_idx[:, None] == jnp.arange(8)[None, :]).astype(
        jnp.float32)

    rows_blk = 32 * sent_len
    pos_slab = jnp.tile(pos_table[:sent_len], (32, 1))
    gb = jnp.concatenate(
        [gamma[None, :], beta[None, :], jnp.zeros((6, hidden), jnp.float32)],
        axis=0)
    out = _tc_finish(gathered.reshape(n_tok, hidden), onehot, pos_slab,
                     combo8, gb, rows_blk)
    return out.reshape(bsz, sent_len, hidden)
